# baseline (speedup 1.0000x reference)
import functools

import jax
import jax.numpy as jnp
from jax import lax
from jax.experimental import pallas as pl
from jax.experimental.pallas import tpu as pltpu
from jax.experimental.pallas import tpu_sc as plsc

NC = 2
NS = 16
CH = 128


@functools.partial(jax.jit, static_argnums=(4, 5, 6))
def _run(feature, src2, dst2, zeros, N, D, n_chunks):
    nup = -(-(N + 1) // (8 * NS)) * (8 * NS)
    zrows = nup // NS

    mesh = plsc.VectorSubcoreMesh(core_axis_name="c", subcore_axis_name="s")

    @functools.partial(
        pl.kernel,
        out_type=jax.ShapeDtypeStruct((NC, nup, D), jnp.float32),
        mesh=mesh,
        scratch_types=[
            pltpu.VMEM_SHARED((nup, D), jnp.float32),
            pltpu.VMEM((n_chunks // 2, CH), jnp.int32),
            pltpu.VMEM((n_chunks // 2, CH), jnp.int32),
            pltpu.VMEM((CH, D), jnp.float32),
            pltpu.VMEM((CH, D), jnp.float32),
            pltpu.SemaphoreType.DMA,
            pltpu.SemaphoreType.DMA,
        ],
    )
    def k(feat_hbm, src_hbm, dst_hbm, zeros_hbm, part_hbm, acc, src_v, dst_v,
          rows_a, rows_b, sem_a, sem_b):
        c = lax.axis_index("c")
        s = lax.axis_index("s")
        wid = s * NC + c
        nh = n_chunks // 2

        def run_half(h, first):
            cb = wid * n_chunks + h * nh
            pltpu.sync_copy(src_hbm.at[pl.ds(cb, nh)], src_v)
            pltpu.sync_copy(dst_hbm.at[pl.ds(cb, nh)], dst_v)
            pltpu.async_copy(feat_hbm.at[src_v.at[0]], rows_a, sem_a)
            pltpu.async_copy(feat_hbm.at[src_v.at[1]], rows_b, sem_b)
            if first:
                pltpu.sync_copy(zeros_hbm, acc.at[pl.ds(s * zrows, zrows)])
                plsc.subcore_barrier()

            def step(j, carry):
                a = 2 * j
                pltpu.make_async_copy(feat_hbm.at[src_v.at[a]], rows_a,
                                      sem_a).wait()
                pltpu.sync_copy(rows_a, acc.at[dst_v.at[a]], add=True)

                @pl.when(a + 2 < nh)
                def _():
                    pltpu.async_copy(feat_hbm.at[src_v.at[a + 2]], rows_a,
                                     sem_a)

                pltpu.make_async_copy(feat_hbm.at[src_v.at[a + 1]], rows_b,
                                      sem_b).wait()
                pltpu.sync_copy(rows_b, acc.at[dst_v.at[a + 1]], add=True)

                @pl.when(a + 3 < nh)
                def _():
                    pltpu.async_copy(feat_hbm.at[src_v.at[a + 3]], rows_b,
                                     sem_b)

                return carry

            lax.fori_loop(0, nh // 2, step, 0)

        run_half(0, True)
        run_half(1, False)
        plsc.subcore_barrier()
        pltpu.sync_copy(acc.at[pl.ds(s * zrows, zrows)],
                        part_hbm.at[c].at[pl.ds(s * zrows, zrows)])

    part = k(feature, src2, dst2, zeros)

    rb = 1000

    def add_body(p_ref, o_ref):
        o_ref[...] = p_ref[0] + p_ref[1]

    return pl.pallas_call(
        add_body,
        grid=(N // rb,),
        in_specs=[pl.BlockSpec((NC, rb, D), lambda i: (0, i, 0))],
        out_specs=pl.BlockSpec((rb, D), lambda i: (i, 0)),
        out_shape=jax.ShapeDtypeStruct((N, D), jnp.float32),
    )(part)


def kernel(feature, edge_index):
    N, D = feature.shape
    E = edge_index.shape[1]
    nw = NC * NS
    n_chunks = -(-(-(-E // nw)) // (8 * CH)) * 8
    EP = n_chunks * CH * nw
    pad = EP - E
    src = jnp.concatenate(
        [edge_index[0].astype(jnp.int32), jnp.zeros((pad,), jnp.int32)])
    dst = jnp.concatenate(
        [edge_index[1].astype(jnp.int32), jnp.full((pad,), N, jnp.int32)])
    src2 = src.reshape(EP // CH, CH)
    dst2 = dst.reshape(EP // CH, CH)
    nup = -(-(N + 1) // (8 * NS)) * (8 * NS)
    zeros = jnp.zeros((nup // NS, D), jnp.float32)
    return _run(feature, src2, dst2, zeros, N, D, n_chunks)
